# baseline (speedup 1.0000x reference)
import functools

import jax
import jax.numpy as jnp
from jax import lax
from jax.experimental import pallas as pl
from jax.experimental.pallas import tpu as pltpu
from jax.experimental.pallas import tpu_sc as plsc

T = 4096
H = 1024
E = 8
HA = H + 256
TM = 256
NT = T // TM + E
CH = 512
NC = T // CH
NW = 32
TPW = T // NW
SCC = 64


def _router_body(x_ref, wg_ref, xg_ref, slot_ref, te_ref, c_ref, oh_ref, lr_ref):
    c = pl.program_id(0)
    x_blk = x_ref[...]
    logits = jnp.dot(x_blk, wg_ref[...], preferred_element_type=jnp.float32)
    m = jnp.max(logits, axis=1, keepdims=True)
    s = jnp.sum(jnp.exp(logits - m), axis=1, keepdims=True)
    gate = 1.0 / s
    lane = lax.broadcasted_iota(jnp.int32, (CH, E), 1)
    idx = jnp.min(jnp.where(logits == m, lane, E), axis=1, keepdims=True)
    oh = (lane == idx).astype(jnp.float32)

    xg_ref[:, :H] = (x_blk * gate).astype(jnp.bfloat16)
    pad_lane = lax.broadcasted_iota(jnp.int32, (CH, HA - H), 1)
    xg_ref[:, H:] = jnp.where(pad_lane == 0, gate, 0.0).astype(jnp.bfloat16)

    r_i = lax.broadcasted_iota(jnp.int32, (CH, CH), 0)
    c_i = lax.broadcasted_iota(jnp.int32, (CH, CH), 1)
    tri = (c_i <= r_i).astype(jnp.float32)
    csum = jnp.dot(tri, oh, preferred_element_type=jnp.float32)
    lr = jnp.sum(csum * oh, axis=1) - 1.0
    lr_ref[pl.ds(c, 1), :] = lr[None, :]
    oh_ref[pl.ds(c * CH, CH), :] = oh
    c_ref[pl.ds(c, 1), :] = jnp.sum(oh, axis=0, keepdims=True)

    @pl.when(c == NC - 1)
    def _finalize():
        cnt = c_ref[...]
        tot = jnp.sum(cnt, axis=0, keepdims=True)
        cpad = jnp.ceil(tot / TM) * TM
        e_r = lax.broadcasted_iota(jnp.int32, (E, E), 0)
        e_c = lax.broadcasted_iota(jnp.int32, (E, E), 1)
        base = jnp.dot(cpad, (e_r < e_c).astype(jnp.float32),
                       preferred_element_type=jnp.float32)
        n_r = lax.broadcasted_iota(jnp.int32, (NC, NC), 0)
        n_c = lax.broadcasted_iota(jnp.int32, (NC, NC), 1)
        strict = (n_c < n_r).astype(jnp.float32)
        pre = jnp.dot(strict, cnt, preferred_element_type=jnp.float32)
        off = base + pre
        for cc in range(NC):
            oh_cc = oh_ref[pl.ds(cc * CH, CH), :]
            off_cc = off[cc:cc + 1, :]
            slot = jnp.sum(oh_cc * off_cc, axis=1)[None, :] + lr_ref[pl.ds(cc, 1), :]
            slot_ref[pl.ds(cc, 1), :] = slot.astype(jnp.int32)
        tb = base / TM
        tid = lax.broadcasted_iota(jnp.int32, (1, 128), 1).astype(jnp.float32)
        te = jnp.zeros((1, 128), jnp.float32)
        for e in range(1, E):
            te = te + (tid >= tb[:, e:e + 1]).astype(jnp.float32)
        nact = jnp.sum(cpad, axis=1, keepdims=True) / TM
        out_lane = lax.broadcasted_iota(jnp.int32, (1, 128), 1)
        te_ref[...] = jnp.where(out_lane == 127, nact, te).astype(jnp.int32)


def _route(x, Wg):
    return pl.pallas_call(
        _router_body,
        grid=(NC,),
        in_specs=[
            pl.BlockSpec((CH, H), lambda c: (c, 0)),
            pl.BlockSpec((H, E), lambda c: (0, 0)),
        ],
        out_specs=[
            pl.BlockSpec((CH, HA), lambda c: (c, 0)),
            pl.BlockSpec((NC, CH), lambda c: (0, 0)),
            pl.BlockSpec((1, 128), lambda c: (0, 0)),
        ],
        out_shape=[
            jax.ShapeDtypeStruct((T, HA), jnp.bfloat16),
            jax.ShapeDtypeStruct((NC, CH), jnp.int32),
            jax.ShapeDtypeStruct((1, 128), jnp.int32),
        ],
        scratch_shapes=[
            pltpu.VMEM((NC, E), jnp.float32),
            pltpu.VMEM((T, E), jnp.float32),
            pltpu.VMEM((NC, CH), jnp.float32),
        ],
    )(x, Wg)


def _dispatch_body(xg_hbm, slot_hbm, xs_hbm, idx_v, rows_v, sem):
    wid = lax.axis_index("s") * 2 + lax.axis_index("c")
    for j in range(TPW // SCC):
        tok0 = wid * TPW + j * SCC
        pltpu.sync_copy(slot_hbm.at[pl.ds(tok0, SCC)], idx_v)
        pltpu.sync_copy(xg_hbm.at[pl.ds(tok0, SCC)], rows_v)
        pltpu.async_copy(rows_v, xs_hbm.at[idx_v], sem).wait()


def _combine_body(ys_hbm, slot_hbm, out_hbm, idx_v, rows_v, sem):
    wid = lax.axis_index("s") * 2 + lax.axis_index("c")
    for j in range(TPW // SCC):
        tok0 = wid * TPW + j * SCC
        pltpu.sync_copy(slot_hbm.at[pl.ds(tok0, SCC)], idx_v)
        pltpu.async_copy(ys_hbm.at[idx_v], rows_v, sem).wait()
        pltpu.sync_copy(rows_v, out_hbm.at[pl.ds(tok0, SCC)])


@functools.lru_cache(maxsize=None)
def _sc_kernels():
    mesh = plsc.VectorSubcoreMesh(core_axis_name="c", subcore_axis_name="s")
    dispatch = pl.kernel(
        _dispatch_body,
        out_type=jax.ShapeDtypeStruct((NT * TM, HA // 2), jnp.int32),
        mesh=mesh,
        scratch_types=[
            pltpu.VMEM((SCC,), jnp.int32),
            pltpu.VMEM((SCC, HA // 2), jnp.int32),
            pltpu.SemaphoreType.DMA,
        ],
    )
    combine = pl.kernel(
        _combine_body,
        out_type=jax.ShapeDtypeStruct((T, H), jnp.float32),
        mesh=mesh,
        scratch_types=[
            pltpu.VMEM((SCC,), jnp.int32),
            pltpu.VMEM((SCC, H), jnp.float32),
            pltpu.SemaphoreType.DMA,
        ],
    )
    return dispatch, combine


def _mm_body(s_ref, xs_ref, we_ref, be_ref, ys_ref):
    i = pl.program_id(0)

    @pl.when(i < s_ref[127])
    def _():
        e = s_ref[i]
        xb = xs_ref[:, :H]
        g = xs_ref[:, H:H + 1].astype(jnp.float32)
        acc = jnp.dot(xb, we_ref[e], preferred_element_type=jnp.float32)
        ys_ref[...] = acc + g * be_ref[e]


def _grouped_mm(sinfo, xs, We_b, be3):
    grid_spec = pltpu.PrefetchScalarGridSpec(
        num_scalar_prefetch=1,
        grid=(NT,),
        in_specs=[
            pl.BlockSpec((TM, HA), lambda i, s: (i, 0)),
            pl.BlockSpec((E, H, H), lambda i, s: (0, 0, 0)),
            pl.BlockSpec((E, 1, H), lambda i, s: (0, 0, 0)),
        ],
        out_specs=pl.BlockSpec((TM, H), lambda i, s: (i, 0)),
    )
    return pl.pallas_call(
        _mm_body,
        grid_spec=grid_spec,
        out_shape=jax.ShapeDtypeStruct((NT * TM, H), jnp.float32),
    )(sinfo, xs, We_b, be3)


def kernel(x, Wg, We, be):
    xg_aug, slot2d, te2d = _route(x, Wg)
    slot = slot2d.reshape(T)
    sinfo = te2d.reshape(128)
    _dispatch, _combine = _sc_kernels()
    xg32 = lax.bitcast_convert_type(xg_aug.reshape(T, HA // 2, 2), jnp.int32)
    xs32 = _dispatch(xg32, slot)
    xs = lax.bitcast_convert_type(xs32, jnp.bfloat16).reshape(NT * TM, HA)
    ys = _grouped_mm(sinfo, xs, We.astype(jnp.bfloat16), be.reshape(E, 1, H))
    return _combine(ys, slot)


# speedup vs baseline: 3.2373x; 3.2373x over previous
import functools

import jax
import jax.numpy as jnp
from jax import lax
from jax.experimental import pallas as pl
from jax.experimental.pallas import tpu as pltpu
from jax.experimental.pallas import tpu_sc as plsc

T = 4096
H = 1024
E = 8
HA = H + 128
TM = 256
NT = T // TM + E
CH = 512
NC = T // CH
NW = 32
TPW = T // NW
SCC = 64


def _router_body(x_ref, wg_ref, xg_ref, slot_ref, te_ref, c_ref, oh_ref, lr_ref):
    c = pl.program_id(0)
    x_blk = x_ref[...]
    logits = jnp.dot(x_blk, wg_ref[...], preferred_element_type=jnp.float32)
    m = jnp.max(logits, axis=1, keepdims=True)
    s = jnp.sum(jnp.exp(logits - m), axis=1, keepdims=True)
    gate = 1.0 / s
    lane = lax.broadcasted_iota(jnp.int32, (CH, E), 1)
    idx = jnp.min(jnp.where(logits == m, lane, E), axis=1, keepdims=True)
    oh = (lane == idx).astype(jnp.float32)

    xg_ref[:, :H] = x_blk * gate
    pad_lane = lax.broadcasted_iota(jnp.int32, (CH, HA - H), 1)
    xg_ref[:, H:] = jnp.where(pad_lane == 0, gate, 0.0)

    r_i = lax.broadcasted_iota(jnp.int32, (CH, CH), 0)
    c_i = lax.broadcasted_iota(jnp.int32, (CH, CH), 1)
    tri = (c_i <= r_i).astype(jnp.float32)
    csum = jnp.dot(tri, oh, preferred_element_type=jnp.float32)
    lr = jnp.sum(csum * oh, axis=1) - 1.0
    lr_ref[pl.ds(c, 1), :] = lr[None, :]
    oh_ref[pl.ds(c * CH, CH), :] = oh
    c_ref[pl.ds(c, 1), :] = jnp.sum(oh, axis=0, keepdims=True)

    @pl.when(c == NC - 1)
    def _finalize():
        cnt = c_ref[...]
        tot = jnp.sum(cnt, axis=0, keepdims=True)
        cpad = jnp.ceil(tot / TM) * TM
        e_r = lax.broadcasted_iota(jnp.int32, (E, E), 0)
        e_c = lax.broadcasted_iota(jnp.int32, (E, E), 1)
        base = jnp.dot(cpad, (e_r < e_c).astype(jnp.float32),
                       preferred_element_type=jnp.float32)
        n_r = lax.broadcasted_iota(jnp.int32, (NC, NC), 0)
        n_c = lax.broadcasted_iota(jnp.int32, (NC, NC), 1)
        strict = (n_c < n_r).astype(jnp.float32)
        pre = jnp.dot(strict, cnt, preferred_element_type=jnp.float32)
        off = base + pre
        for cc in range(NC):
            oh_cc = oh_ref[pl.ds(cc * CH, CH), :]
            off_cc = off[cc:cc + 1, :]
            slot = jnp.sum(oh_cc * off_cc, axis=1)[None, :] + lr_ref[pl.ds(cc, 1), :]
            slot_ref[pl.ds(cc, 1), :] = slot.astype(jnp.int32)
        tb = base / TM
        tid = lax.broadcasted_iota(jnp.int32, (1, 128), 1).astype(jnp.float32)
        te = jnp.zeros((1, 128), jnp.float32)
        for e in range(1, E):
            te = te + (tid >= tb[:, e:e + 1]).astype(jnp.float32)
        nact = jnp.sum(cpad, axis=1, keepdims=True) / TM
        out_lane = lax.broadcasted_iota(jnp.int32, (1, 128), 1)
        te_ref[...] = jnp.where(out_lane == 127, nact, te).astype(jnp.int32)


def _route(x, Wg):
    return pl.pallas_call(
        _router_body,
        grid=(NC,),
        in_specs=[
            pl.BlockSpec((CH, H), lambda c: (c, 0)),
            pl.BlockSpec((H, E), lambda c: (0, 0)),
        ],
        out_specs=[
            pl.BlockSpec((CH, HA), lambda c: (c, 0)),
            pl.BlockSpec((NC, CH), lambda c: (0, 0)),
            pl.BlockSpec((1, 128), lambda c: (0, 0)),
        ],
        out_shape=[
            jax.ShapeDtypeStruct((T, HA), jnp.float32),
            jax.ShapeDtypeStruct((NC, CH), jnp.int32),
            jax.ShapeDtypeStruct((1, 128), jnp.int32),
        ],
        scratch_shapes=[
            pltpu.VMEM((NC, E), jnp.float32),
            pltpu.VMEM((T, E), jnp.float32),
            pltpu.VMEM((NC, CH), jnp.float32),
        ],
    )(x, Wg)


def _dispatch_body(xg_hbm, slot_hbm, xs_hbm, idx_v, rows_v, sem):
    wid = lax.axis_index("s") * 2 + lax.axis_index("c")
    for j in range(TPW // SCC):
        tok0 = wid * TPW + j * SCC
        pltpu.sync_copy(slot_hbm.at[pl.ds(tok0, SCC)], idx_v)
        pltpu.sync_copy(xg_hbm.at[pl.ds(tok0, SCC)], rows_v)
        pltpu.async_copy(rows_v, xs_hbm.at[idx_v], sem).wait()


def _combine_body(ys_hbm, slot_hbm, out_hbm, idx_v, rows_v, sem):
    wid = lax.axis_index("s") * 2 + lax.axis_index("c")
    for j in range(TPW // SCC):
        tok0 = wid * TPW + j * SCC
        pltpu.sync_copy(slot_hbm.at[pl.ds(tok0, SCC)], idx_v)
        pltpu.async_copy(ys_hbm.at[idx_v], rows_v, sem).wait()
        pltpu.sync_copy(rows_v, out_hbm.at[pl.ds(tok0, SCC)])


@functools.lru_cache(maxsize=None)
def _sc_kernels():
    mesh = plsc.VectorSubcoreMesh(core_axis_name="c", subcore_axis_name="s")
    dispatch = pl.kernel(
        _dispatch_body,
        out_type=jax.ShapeDtypeStruct((NT * TM, HA), jnp.float32),
        mesh=mesh,
        scratch_types=[
            pltpu.VMEM((SCC,), jnp.int32),
            pltpu.VMEM((SCC, HA), jnp.float32),
            pltpu.SemaphoreType.DMA,
        ],
    )
    combine = pl.kernel(
        _combine_body,
        out_type=jax.ShapeDtypeStruct((T, H), jnp.float32),
        mesh=mesh,
        scratch_types=[
            pltpu.VMEM((SCC,), jnp.int32),
            pltpu.VMEM((SCC, H), jnp.float32),
            pltpu.SemaphoreType.DMA,
        ],
    )
    return dispatch, combine


def _mm_body(s_ref, xs_ref, we_ref, be_ref, ys_ref):
    i = pl.program_id(0)

    @pl.when(i < s_ref[127])
    def _():
        e = s_ref[i]
        xb = xs_ref[:, :H].astype(jnp.bfloat16)
        g = xs_ref[:, H:H + 1]
        acc = jnp.dot(xb, we_ref[e], preferred_element_type=jnp.float32)
        ys_ref[...] = acc + g * be_ref[e]


def _grouped_mm(sinfo, xs, We_b, be3):
    grid_spec = pltpu.PrefetchScalarGridSpec(
        num_scalar_prefetch=1,
        grid=(NT,),
        in_specs=[
            pl.BlockSpec((TM, HA), lambda i, s: (i, 0)),
            pl.BlockSpec((E, H, H), lambda i, s: (0, 0, 0)),
            pl.BlockSpec((E, 1, H), lambda i, s: (0, 0, 0)),
        ],
        out_specs=pl.BlockSpec((TM, H), lambda i, s: (i, 0)),
    )
    return pl.pallas_call(
        _mm_body,
        grid_spec=grid_spec,
        out_shape=jax.ShapeDtypeStruct((NT * TM, H), jnp.float32),
    )(sinfo, xs, We_b, be3)


def kernel(x, Wg, We, be):
    xg_aug, slot2d, te2d = _route(x, Wg)
    slot = slot2d.reshape(T)
    sinfo = te2d.reshape(128)
    _dispatch, _combine = _sc_kernels()
    xs = _dispatch(xg_aug, slot)
    ys = _grouped_mm(sinfo, xs, We.astype(jnp.bfloat16), be.reshape(E, 1, H))
    return _combine(ys, slot)


# speedup vs baseline: 3.7319x; 1.1528x over previous
import functools

import jax
import jax.numpy as jnp
from jax import lax
from jax.experimental import pallas as pl
from jax.experimental.pallas import tpu as pltpu
from jax.experimental.pallas import tpu_sc as plsc

T = 4096
H = 1024
E = 8
GW = 128
TM = 256
NT = T // TM + E
CH = 512
NC = T // CH
NW = 32
TPW = T // NW
SCC = 64


def _router_body(x_ref, wg_ref, gc_ref, slot_ref, te_ref, c_ref, oh_ref, lr_ref):
    c = pl.program_id(0)
    x_blk = x_ref[...]
    logits = jnp.dot(x_blk, wg_ref[...], preferred_element_type=jnp.float32)
    m = jnp.max(logits, axis=1, keepdims=True)
    s = jnp.sum(jnp.exp(logits - m), axis=1, keepdims=True)
    gate = 1.0 / s
    lane = lax.broadcasted_iota(jnp.int32, (CH, E), 1)
    idx = jnp.min(jnp.where(logits == m, lane, E), axis=1, keepdims=True)
    oh = (lane == idx).astype(jnp.float32)

    gc_lane = lax.broadcasted_iota(jnp.int32, (CH, GW), 1)
    gc_ref[...] = jnp.where(gc_lane == 0, gate, 0.0)

    r_i = lax.broadcasted_iota(jnp.int32, (CH, CH), 0)
    c_i = lax.broadcasted_iota(jnp.int32, (CH, CH), 1)
    tri = (c_i <= r_i).astype(jnp.float32)
    csum = jnp.dot(tri, oh, preferred_element_type=jnp.float32)
    lr = jnp.sum(csum * oh, axis=1) - 1.0
    lr_ref[pl.ds(c, 1), :] = lr[None, :]
    oh_ref[pl.ds(c * CH, CH), :] = oh
    c_ref[pl.ds(c, 1), :] = jnp.sum(oh, axis=0, keepdims=True)

    @pl.when(c == NC - 1)
    def _finalize():
        cnt = c_ref[...]
        tot = jnp.sum(cnt, axis=0, keepdims=True)
        cpad = jnp.ceil(tot / TM) * TM
        e_r = lax.broadcasted_iota(jnp.int32, (E, E), 0)
        e_c = lax.broadcasted_iota(jnp.int32, (E, E), 1)
        base = jnp.dot(cpad, (e_r < e_c).astype(jnp.float32),
                       preferred_element_type=jnp.float32)
        n_r = lax.broadcasted_iota(jnp.int32, (NC, NC), 0)
        n_c = lax.broadcasted_iota(jnp.int32, (NC, NC), 1)
        strict = (n_c < n_r).astype(jnp.float32)
        pre = jnp.dot(strict, cnt, preferred_element_type=jnp.float32)
        off = base + pre
        for cc in range(NC):
            oh_cc = oh_ref[pl.ds(cc * CH, CH), :]
            off_cc = off[cc:cc + 1, :]
            slot = jnp.sum(oh_cc * off_cc, axis=1)[None, :] + lr_ref[pl.ds(cc, 1), :]
            slot_ref[pl.ds(cc, 1), :] = slot.astype(jnp.int32)
        tb = base / TM
        tid = lax.broadcasted_iota(jnp.int32, (1, 128), 1).astype(jnp.float32)
        te = jnp.zeros((1, 128), jnp.float32)
        for e in range(1, E):
            te = te + (tid >= tb[:, e:e + 1]).astype(jnp.float32)
        nact = jnp.sum(cpad, axis=1, keepdims=True) / TM
        out_lane = lax.broadcasted_iota(jnp.int32, (1, 128), 1)
        te_ref[...] = jnp.where(out_lane == 127, nact, te).astype(jnp.int32)


def _route(x, Wg):
    return pl.pallas_call(
        _router_body,
        grid=(NC,),
        in_specs=[
            pl.BlockSpec((CH, H), lambda c: (c, 0)),
            pl.BlockSpec((H, E), lambda c: (0, 0)),
        ],
        out_specs=[
            pl.BlockSpec((CH, GW), lambda c: (c, 0)),
            pl.BlockSpec((NC, CH), lambda c: (0, 0)),
            pl.BlockSpec((1, 128), lambda c: (0, 0)),
        ],
        out_shape=[
            jax.ShapeDtypeStruct((T, GW), jnp.float32),
            jax.ShapeDtypeStruct((NC, CH), jnp.int32),
            jax.ShapeDtypeStruct((1, 128), jnp.int32),
        ],
        scratch_shapes=[
            pltpu.VMEM((NC, E), jnp.float32),
            pltpu.VMEM((T, E), jnp.float32),
            pltpu.VMEM((NC, CH), jnp.float32),
        ],
    )(x, Wg)


def _dispatch_body(x_hbm, gc_hbm, slot_hbm, xs_hbm, gp_hbm,
                   idx_v, rows_v, grows_v, sem):
    wid = lax.axis_index("s") * 2 + lax.axis_index("c")
    for j in range(TPW // SCC):
        tok0 = wid * TPW + j * SCC
        pltpu.sync_copy(slot_hbm.at[pl.ds(tok0, SCC)], idx_v)
        pltpu.sync_copy(x_hbm.at[pl.ds(tok0, SCC)], rows_v)
        pltpu.sync_copy(gc_hbm.at[pl.ds(tok0, SCC)], grows_v)
        pltpu.async_copy(rows_v, xs_hbm.at[idx_v], sem).wait()
        pltpu.async_copy(grows_v, gp_hbm.at[idx_v], sem).wait()


def _combine_body(ys_hbm, slot_hbm, out_hbm, idx_v, rows_v, sem):
    wid = lax.axis_index("s") * 2 + lax.axis_index("c")
    for j in range(TPW // SCC):
        tok0 = wid * TPW + j * SCC
        pltpu.sync_copy(slot_hbm.at[pl.ds(tok0, SCC)], idx_v)
        pltpu.async_copy(ys_hbm.at[idx_v], rows_v, sem).wait()
        pltpu.sync_copy(rows_v, out_hbm.at[pl.ds(tok0, SCC)])


@functools.lru_cache(maxsize=None)
def _sc_kernels():
    mesh = plsc.VectorSubcoreMesh(core_axis_name="c", subcore_axis_name="s")
    dispatch = pl.kernel(
        _dispatch_body,
        out_type=(
            jax.ShapeDtypeStruct((NT * TM, H), jnp.float32),
            jax.ShapeDtypeStruct((NT * TM, GW), jnp.float32),
        ),
        mesh=mesh,
        scratch_types=[
            pltpu.VMEM((SCC,), jnp.int32),
            pltpu.VMEM((SCC, H), jnp.float32),
            pltpu.VMEM((SCC, GW), jnp.float32),
            pltpu.SemaphoreType.DMA,
        ],
    )
    combine = pl.kernel(
        _combine_body,
        out_type=jax.ShapeDtypeStruct((T, H), jnp.float32),
        mesh=mesh,
        scratch_types=[
            pltpu.VMEM((SCC,), jnp.int32),
            pltpu.VMEM((SCC, H), jnp.float32),
            pltpu.SemaphoreType.DMA,
        ],
    )
    return dispatch, combine


def _mm_body(s_ref, xs_ref, gp_ref, we_ref, be_ref, ys_ref, web_ref):
    i = pl.program_id(0)

    @pl.when(i < s_ref[127])
    def _():
        e = s_ref[i]
        first = i == 0
        changed = jnp.logical_or(first, s_ref[jnp.maximum(i - 1, 0)] != e)

        @pl.when(changed)
        def _convert():
            web_ref[e] = we_ref[e].astype(jnp.bfloat16)

        xb = xs_ref[...].astype(jnp.bfloat16)
        g = gp_ref[:, 0:1]
        acc = jnp.dot(xb, web_ref[e], preferred_element_type=jnp.float32)
        ys_ref[...] = (acc + be_ref[e]) * g


def _grouped_mm(sinfo, xs, gpad, We, be3):
    def _tile(i, s):
        return (jnp.minimum(i, s[127] - 1), 0)

    grid_spec = pltpu.PrefetchScalarGridSpec(
        num_scalar_prefetch=1,
        grid=(NT,),
        in_specs=[
            pl.BlockSpec((TM, H), _tile),
            pl.BlockSpec((TM, GW), _tile),
            pl.BlockSpec((E, H, H), lambda i, s: (0, 0, 0)),
            pl.BlockSpec((E, 1, H), lambda i, s: (0, 0, 0)),
        ],
        out_specs=pl.BlockSpec((TM, H), _tile),
        scratch_shapes=[pltpu.VMEM((E, H, H), jnp.bfloat16)],
    )
    return pl.pallas_call(
        _mm_body,
        grid_spec=grid_spec,
        out_shape=jax.ShapeDtypeStruct((NT * TM, H), jnp.float32),
    )(sinfo, xs, gpad, We, be3)


def kernel(x, Wg, We, be):
    gcol, slot2d, te2d = _route(x, Wg)
    slot = slot2d.reshape(T)
    sinfo = te2d.reshape(128)
    _dispatch, _combine = _sc_kernels()
    xs, gpad = _dispatch(x, gcol, slot)
    ys = _grouped_mm(sinfo, xs, gpad, We, be.reshape(E, 1, H))
    return _combine(ys, slot)
